# baseline (speedup 1.0000x reference)
import functools

import jax
import jax.numpy as jnp
from jax import lax
from jax.experimental import pallas as pl
from jax.experimental.pallas import tpu as pltpu
from jax.experimental.pallas import tpu_sc as plsc

_HID = 64
_OUT = 32
_HEADS = 4

_BM = 1024


def _pad_rows(x, bm=_BM):
    m = x.shape[0]
    mp = ((m + bm - 1) // bm) * bm
    if mp == m:
        return x
    return jnp.pad(x, ((0, mp - m), (0, 0)))


def _mm_body(x_ref, w_ref, b_ref, o_ref, *, act):
    acc = jnp.dot(x_ref[...], w_ref[...], preferred_element_type=jnp.float32)
    acc = acc + b_ref[...]
    if act == "relu":
        acc = jnp.maximum(acc, 0.0)
    o_ref[...] = acc


def _mm(x, w, b, act=None):
    m = x.shape[0]
    xp = _pad_rows(x)
    mp, k = xp.shape
    n = w.shape[1]
    out = pl.pallas_call(
        functools.partial(_mm_body, act=act),
        grid=(mp // _BM,),
        in_specs=[
            pl.BlockSpec((_BM, k), lambda i: (i, 0)),
            pl.BlockSpec((k, n), lambda i: (0, 0)),
            pl.BlockSpec((1, n), lambda i: (0, 0)),
        ],
        out_specs=pl.BlockSpec((_BM, n), lambda i: (i, 0)),
        out_shape=jax.ShapeDtypeStruct((mp, n), jnp.float32),
    )(xp, w, b.reshape(1, n))
    return out[:m]


def _gatproj_body(h_ref, w_ref, avs_ref, avd_ref, s_ref, *out_refs, outs):
    hs = jnp.dot(h_ref[...], w_ref[...], preferred_element_type=jnp.float32)
    i = 0
    if "hs" in outs:
        out_refs[i][...] = hs
        i += 1
    if "es" in outs:
        out_refs[i][...] = jnp.dot(hs * avs_ref[...], s_ref[...],
                                   preferred_element_type=jnp.float32)
        i += 1
    if "ed" in outs:
        out_refs[i][...] = jnp.dot(hs * avd_ref[...], s_ref[...],
                                   preferred_element_type=jnp.float32)
        i += 1


def _gat_proj(h, w, a_s, a_d, outs):
    m = h.shape[0]
    hp = _pad_rows(h)
    mp, k = hp.shape
    heads_ch = w.shape[1]
    heads = a_s.shape[0]
    ch = heads_ch // heads
    avs = a_s.reshape(1, heads_ch)
    avd = a_d.reshape(1, heads_ch)
    sel = jnp.zeros((heads_ch, 16), jnp.float32)
    sel = sel.at[jnp.arange(heads_ch), jnp.arange(heads_ch) // ch].set(1.0)
    shapes = {"hs": (mp, heads_ch), "es": (mp, 16), "ed": (mp, 16)}
    out_shape = [jax.ShapeDtypeStruct(shapes[o], jnp.float32) for o in outs]
    out_specs = [pl.BlockSpec((_BM, shapes[o][1]), lambda i: (i, 0))
                 for o in outs]
    res = pl.pallas_call(
        functools.partial(_gatproj_body, outs=outs),
        grid=(mp // _BM,),
        in_specs=[
            pl.BlockSpec((_BM, k), lambda i: (i, 0)),
            pl.BlockSpec((k, heads_ch), lambda i: (0, 0)),
            pl.BlockSpec((1, heads_ch), lambda i: (0, 0)),
            pl.BlockSpec((1, heads_ch), lambda i: (0, 0)),
            pl.BlockSpec((heads_ch, 16), lambda i: (0, 0)),
        ],
        out_specs=out_specs,
        out_shape=out_shape,
    )(hp, w, avs, avd, sel)
    if not isinstance(res, (list, tuple)):
        res = [res]
    return {o: r[:m] for o, r in zip(outs, res)}


_CH = 2048
_GK = 128
_NC, _NS = 2, 16


def _pick_tile(n_dst, w):
    scratch_words = 4 * _CH + 288 + 3 * _GK + _GK * w + _GK * 16 + 16 * w
    budget = 2097151 - _NS * scratch_words - 65536
    acc_rows = (budget // (w * 256)) * 256
    tile = acc_rows - 16
    n_tiles = max(2, 2 * (-(-n_dst // (2 * tile))))
    per = -(-n_dst // n_tiles)
    acc_rows = 256 * (-(-(per + 16) // 256))
    return acc_rows - 16, n_tiles, acc_rows


def _sc_agg_body(hsx, edp, srcp, dstp, out, srcv, dstv, sbuf, dbuf, gidx,
                 lidx, eidx, rows, edr, zbuf, sem1, sem2, semr, seme, *,
                 heads, ch, tile, acc_rows, tiles_per_core, n_chunk_iters,
                 acc):
    hc = heads * ch
    w = hc + 16
    c = lax.axis_index("c")
    s = lax.axis_index("s")
    rows_per_sub = acc_rows // _NS
    nz = rows_per_sub // 16
    row0 = s * rows_per_sub

    zv = jnp.zeros((16,), jnp.float32)
    for r in range(16):
        for j in range(w // 16):
            zbuf[r, pl.ds(j * 16, 16)] = zv

    def edge_body(e, _):
        esv = rows[e, pl.ds(hc, 16)]
        edv = edr[e, :]
        x = esv + edv
        w16 = jnp.exp(jnp.where(x > 0.0, x, 0.2 * x))
        rows[e, pl.ds(hc, 16)] = w16
        for h in range(heads):
            bh = w16.at[jnp.full((16,), h, jnp.int32)].get(
                mode="promise_in_bounds")
            for j in range(ch // 16):
                col = h * ch + j * 16
                rows[e, pl.ds(col, 16)] = rows[e, pl.ds(col, 16)] * bh
        return 0

    def group_body(g, lo):
        gbase = g * _GK
        for j in range(_GK // 16):
            gidx[pl.ds(j * 16, 16)] = sbuf[pl.ds(gbase + j * 16, 16)]
            li = dbuf[pl.ds(gbase + j * 16, 16)]
            lidx[pl.ds(j * 16, 16)] = li
            eidx[pl.ds(j * 16, 16)] = li + lo
        cpr = pltpu.async_copy(hsx.at[gidx], rows, semr)
        cpe = pltpu.async_copy(edp.at[eidx], edr, seme)
        cpr.wait()
        cpe.wait()
        lax.fori_loop(0, _GK, edge_body, 0, unroll=2)
        pltpu.sync_copy(rows, acc.at[lidx], add=True)
        return lo

    def chunk_body(kc, lo):
        base = (kc * _NS + s) * _CH
        cp1 = pltpu.async_copy(srcp.at[pl.ds(base, _CH)], srcv, sem1)
        cp2 = pltpu.async_copy(dstp.at[pl.ds(base, _CH)], dstv, sem2)
        cp1.wait()
        cp2.wait()

        def comp_body(g, cnt):
            s16 = srcv[pl.ds(g * 16, 16)]
            dl = dstv[pl.ds(g * 16, 16)] - lo
            m = (dl >= 0) & (dl < tile)
            mi = jnp.where(m, jnp.int32(1), jnp.int32(0))
            pos = plsc.cumsum(mi) - 1 + cnt
            plsc.store_scatter(sbuf, [pos], s16, mask=m)
            plsc.store_scatter(dbuf, [pos], dl, mask=m)
            return cnt + jnp.sum(mi)

        cnt = lax.fori_loop(0, _CH // 16, comp_body, 0)
        z16 = jnp.zeros((16,), jnp.int32)
        t16 = jnp.full((16,), tile, jnp.int32)
        for j in range(_GK // 16):
            sbuf[pl.ds(cnt + j * 16, 16)] = z16
            dbuf[pl.ds(cnt + j * 16, 16)] = t16
        ngroups = (cnt + _GK - 1) // _GK
        lax.fori_loop(0, ngroups, group_body, lo)
        return lo

    def tile_body(tt, _):
        lo = (tt * _NC + c) * tile
        for r in range(nz):
            pltpu.sync_copy(zbuf, acc.at[pl.ds(row0 + r * 16, 16)])
        plsc.subcore_barrier()
        lax.fori_loop(0, n_chunk_iters, chunk_body, lo)
        plsc.subcore_barrier()
        outb = (tt * _NC + c) * acc_rows + row0
        for r in range(nz):
            pltpu.sync_copy(acc.at[pl.ds(row0 + r * 16, 16)],
                            out.at[pl.ds(outb + r * 16, 16)])
        plsc.subcore_barrier()
        return 0

    lax.fori_loop(0, tiles_per_core, tile_body, 0)


def _aggregate(hs, es, ed, src, dst, n_dst, heads, ch):
    hc = heads * ch
    w = hc + 16
    hsx = jnp.concatenate([hs, es], axis=1)
    tile, n_tiles, acc_rows = _pick_tile(n_dst, w)
    tiles_per_core = n_tiles // _NC
    edp = jnp.zeros((n_tiles * tile + 16, 16), jnp.float32).at[:n_dst].set(ed)

    e = src.shape[0]
    ep = -(-e // (_CH * _NS)) * (_CH * _NS)
    srcp = jnp.concatenate([src, jnp.zeros((ep - e,), jnp.int32)])
    dstp = jnp.concatenate(
        [dst, jnp.full((ep - e,), 1 << 30, jnp.int32)])
    n_chunk_iters = ep // (_CH * _NS)

    mesh = plsc.VectorSubcoreMesh(core_axis_name="c", subcore_axis_name="s")
    body = functools.partial(
        _sc_agg_body, heads=heads, ch=ch, tile=tile, acc_rows=acc_rows,
        tiles_per_core=tiles_per_core, n_chunk_iters=n_chunk_iters)

    def run(hsx_a, edp_a, srcp_a, dstp_a):
        def wrapped(hsx_r, edp_r, srcp_r, dstp_r, out_r, *scr):
            acc = scr[-1]
            body(hsx_r, edp_r, srcp_r, dstp_r, out_r, *scr[:-1], acc=acc)
        return pl.kernel(
            wrapped,
            out_type=jax.ShapeDtypeStruct((n_tiles * acc_rows, w),
                                          jnp.float32),
            mesh=mesh,
            compiler_params=pltpu.CompilerParams(
                needs_layout_passes=False, use_tc_tiling_on_sc=False),
            scratch_types=[
                pltpu.VMEM((_CH,), jnp.int32),
                pltpu.VMEM((_CH,), jnp.int32),
                pltpu.VMEM((_CH + 144,), jnp.int32),
                pltpu.VMEM((_CH + 144,), jnp.int32),
                pltpu.VMEM((_GK,), jnp.int32),
                pltpu.VMEM((_GK,), jnp.int32),
                pltpu.VMEM((_GK,), jnp.int32),
                pltpu.VMEM((_GK, w), jnp.float32),
                pltpu.VMEM((_GK, 16), jnp.float32),
                pltpu.VMEM((16, w), jnp.float32),
                pltpu.SemaphoreType.DMA,
                pltpu.SemaphoreType.DMA,
                pltpu.SemaphoreType.DMA,
                pltpu.SemaphoreType.DMA,
                pltpu.VMEM_SHARED((acc_rows, w), jnp.float32),
            ],
        )(hsx_a, edp_a, srcp_a, dstp_a)

    num = run(hsx, edp, srcp, dstp)
    num = num.reshape(n_tiles, acc_rows, w)[:, :tile]
    num = num.reshape(n_tiles * tile, w)[:n_dst]
    return num[:, :hc], num[:, hc:hc + heads]


def _finish(num, den, b, heads, ch):
    den = jnp.where(den > 0, den, 1.0)
    o = num.reshape(-1, heads, ch) / den[:, :, None]
    return o.mean(axis=1) + b


def _gat_edge(h_src, h_dst, src, dst, w1, a_s, a_d, b, heads, ch, same):
    if same:
        p = _gat_proj(h_src, w1, a_s, a_d, ("hs", "es", "ed"))
        hs, es, ed = p["hs"], p["es"], p["ed"]
    else:
        p = _gat_proj(h_src, w1, a_s, a_d, ("hs", "es"))
        hs, es = p["hs"], p["es"]
        ed = _gat_proj(h_dst, w1, a_s, a_d, ("ed",))["ed"]
    num, den = _aggregate(hs, es, ed, src, dst, h_dst.shape[0], heads, ch)
    return _finish(num, den, b, heads, ch)


def kernel(x_user, x_product, x_category, ei_prefers, ei_similar, ei_belongs, Wp_user, bp_user, Wd_user, bd_user, Wo_user, bo_user, Wp_product, bp_product, Wd_product, bd_product, Wo_product, bo_product, Wp_category, bp_category, Wd_category, bd_category, Wo_category, bo_category, W1_prefers, as1_prefers, ad1_prefers, b1_prefers, W2_prefers, as2_prefers, ad2_prefers, b2_prefers, W1_similar, as1_similar, ad1_similar, b1_similar, W2_similar, as2_similar, ad2_similar, b2_similar, W1_belongs, as1_belongs, ad1_belongs, b1_belongs, W2_belongs, as2_belongs, ad2_belongs, b2_belongs):
    n_prod = x_product.shape[0]
    lp = jnp.arange(n_prod, dtype=ei_similar.dtype)
    sim_src = jnp.concatenate([ei_similar[0], lp])
    sim_dst = jnp.concatenate([ei_similar[1], lp])

    h_u = _mm(x_user, Wp_user, bp_user, act="relu")
    h_p = _mm(x_product, Wp_product, bp_product, act="relu")
    h_c = _mm(x_category, Wp_category, bp_category, act="relu")

    o_pref = _gat_edge(h_u, h_p, ei_prefers[0], ei_prefers[1],
                       W1_prefers, as1_prefers, ad1_prefers, b1_prefers,
                       _HEADS, _HID, False)
    o_sim = _gat_edge(h_p, h_p, sim_src, sim_dst,
                      W1_similar, as1_similar, ad1_similar, b1_similar,
                      _HEADS, _HID, True)
    o_bel = _gat_edge(h_p, h_c, ei_belongs[0], ei_belongs[1],
                      W1_belongs, as1_belongs, ad1_belongs, b1_belongs,
                      _HEADS, _HID, False)
    h_p1 = jax.nn.relu(0.5 * (o_pref + o_sim))
    h_c1 = jax.nn.relu(o_bel)

    o_pref2 = _gat_edge(h_u, h_p1, ei_prefers[0], ei_prefers[1],
                        W2_prefers, as2_prefers, ad2_prefers, b2_prefers,
                        1, _OUT, False)
    o_sim2 = _gat_edge(h_p1, h_p1, sim_src, sim_dst,
                       W2_similar, as2_similar, ad2_similar, b2_similar,
                       1, _OUT, True)
    o_bel2 = _gat_edge(h_p1, h_c1, ei_belongs[0], ei_belongs[1],
                       W2_belongs, as2_belongs, ad2_belongs, b2_belongs,
                       1, _OUT, False)
    h_p2 = 0.5 * (o_pref2 + o_sim2)
    h_c2 = o_bel2

    wu = Wd_user @ Wo_user
    bu = bd_user @ Wo_user + bo_user
    out_u = _mm(h_u, wu, bu)
    out_p = _mm(h_p2, Wo_product, bo_product)
    out_c = _mm(h_c2, Wo_category, bo_category)
    return jnp.concatenate([out_u, out_p, out_c], axis=0)


# speedup vs baseline: 1.7641x; 1.7641x over previous
import functools

import jax
import jax.numpy as jnp
from jax import lax
from jax.experimental import pallas as pl
from jax.experimental.pallas import tpu as pltpu
from jax.experimental.pallas import tpu_sc as plsc

_HID = 64
_OUT = 32
_HEADS = 4

_BM = 1024


def _pad_rows(x, bm=_BM):
    m = x.shape[0]
    mp = ((m + bm - 1) // bm) * bm
    if mp == m:
        return x
    return jnp.pad(x, ((0, mp - m), (0, 0)))


def _mm_body(x_ref, w_ref, b_ref, o_ref, *, act):
    acc = jnp.dot(x_ref[...], w_ref[...], preferred_element_type=jnp.float32)
    acc = acc + b_ref[...]
    if act == "relu":
        acc = jnp.maximum(acc, 0.0)
    o_ref[...] = acc


def _mm(x, w, b, act=None):
    m = x.shape[0]
    xp = _pad_rows(x)
    mp, k = xp.shape
    n = w.shape[1]
    out = pl.pallas_call(
        functools.partial(_mm_body, act=act),
        grid=(mp // _BM,),
        in_specs=[
            pl.BlockSpec((_BM, k), lambda i: (i, 0)),
            pl.BlockSpec((k, n), lambda i: (0, 0)),
            pl.BlockSpec((1, n), lambda i: (0, 0)),
        ],
        out_specs=pl.BlockSpec((_BM, n), lambda i: (i, 0)),
        out_shape=jax.ShapeDtypeStruct((mp, n), jnp.float32),
    )(xp, w, b.reshape(1, n))
    return out[:m]


def _gatproj_body(h_ref, w_ref, avs_ref, avd_ref, s_ref, *out_refs, outs):
    hs = jnp.dot(h_ref[...], w_ref[...], preferred_element_type=jnp.float32)
    i = 0
    if "hs" in outs:
        out_refs[i][...] = hs
        i += 1
    if "es" in outs:
        out_refs[i][...] = jnp.dot(hs * avs_ref[...], s_ref[...],
                                   preferred_element_type=jnp.float32)
        i += 1
    if "ed" in outs:
        out_refs[i][...] = jnp.dot(hs * avd_ref[...], s_ref[...],
                                   preferred_element_type=jnp.float32)
        i += 1


def _gat_proj(h, w, a_s, a_d, outs):
    m = h.shape[0]
    hp = _pad_rows(h)
    mp, k = hp.shape
    heads_ch = w.shape[1]
    heads = a_s.shape[0]
    ch = heads_ch // heads
    avs = a_s.reshape(1, heads_ch)
    avd = a_d.reshape(1, heads_ch)
    sel = jnp.zeros((heads_ch, 16), jnp.float32)
    sel = sel.at[jnp.arange(heads_ch), jnp.arange(heads_ch) // ch].set(1.0)
    shapes = {"hs": (mp, heads_ch), "es": (mp, 16), "ed": (mp, 16)}
    out_shape = [jax.ShapeDtypeStruct(shapes[o], jnp.float32) for o in outs]
    out_specs = [pl.BlockSpec((_BM, shapes[o][1]), lambda i: (i, 0))
                 for o in outs]
    res = pl.pallas_call(
        functools.partial(_gatproj_body, outs=outs),
        grid=(mp // _BM,),
        in_specs=[
            pl.BlockSpec((_BM, k), lambda i: (i, 0)),
            pl.BlockSpec((k, heads_ch), lambda i: (0, 0)),
            pl.BlockSpec((1, heads_ch), lambda i: (0, 0)),
            pl.BlockSpec((1, heads_ch), lambda i: (0, 0)),
            pl.BlockSpec((heads_ch, 16), lambda i: (0, 0)),
        ],
        out_specs=out_specs,
        out_shape=out_shape,
    )(hp, w, avs, avd, sel)
    if not isinstance(res, (list, tuple)):
        res = [res]
    return {o: r[:m] for o, r in zip(outs, res)}


_CH = 2048
_GK = 64
_NC, _NS = 2, 16


def _pick_tile(n_dst, w):
    scratch_words = 4 * _CH + 288 + 3 * _GK + _GK * w + _GK * 16 + 16 * w
    budget = 2097151 - _NS * scratch_words - 65536
    acc_rows = (budget // (w * 256)) * 256
    tile = acc_rows - 16
    n_tiles = max(2, 2 * (-(-n_dst // (2 * tile))))
    per = -(-n_dst // n_tiles)
    acc_rows = 256 * (-(-(per + 16) // 256))
    return acc_rows - 16, n_tiles, acc_rows


def _sc_agg_body(hsx, edp, srcp, dstp, out, srcv, dstv, sbuf, dbuf, gidx,
                 lidx, eidx, rows, edr, zbuf, sem1, sem2, semr, seme, *,
                 heads, ch, tile, acc_rows, tiles_per_core, n_chunk_iters,
                 acc):
    hc = heads * ch
    w = hc + 16
    c = lax.axis_index("c")
    s = lax.axis_index("s")
    rows_per_sub = acc_rows // _NS
    nz = rows_per_sub // 16
    row0 = s * rows_per_sub

    zv = jnp.zeros((16,), jnp.float32)
    for r in range(16):
        for j in range(w // 16):
            zbuf[r, pl.ds(j * 16, 16)] = zv

    def edge_body(e, _):
        esv = rows[e, pl.ds(hc, 16)]
        edv = edr[e, :]
        x = esv + edv
        w16 = jnp.exp(jnp.where(x > 0.0, x, 0.2 * x))
        rows[e, pl.ds(hc, 16)] = w16
        for h in range(heads):
            bh = w16.at[jnp.full((16,), h, jnp.int32)].get(
                mode="promise_in_bounds")
            for j in range(ch // 16):
                col = h * ch + j * 16
                rows[e, pl.ds(col, 16)] = rows[e, pl.ds(col, 16)] * bh
        return 0

    def group_body(g, lo):
        gbase = g * _GK
        for j in range(_GK // 16):
            gidx[pl.ds(j * 16, 16)] = sbuf[pl.ds(gbase + j * 16, 16)]
            li = dbuf[pl.ds(gbase + j * 16, 16)]
            lidx[pl.ds(j * 16, 16)] = li
            eidx[pl.ds(j * 16, 16)] = li + lo
        cpr = pltpu.async_copy(hsx.at[gidx], rows, semr)
        cpe = pltpu.async_copy(edp.at[eidx], edr, seme)
        cpr.wait()
        cpe.wait()
        lax.fori_loop(0, _GK, edge_body, 0)
        pltpu.sync_copy(rows, acc.at[lidx], add=True)
        return lo

    def chunk_body(kc, lo):
        base = (kc * _NS + s) * _CH
        cp1 = pltpu.async_copy(srcp.at[pl.ds(base, _CH)], srcv, sem1)
        cp2 = pltpu.async_copy(dstp.at[pl.ds(base, _CH)], dstv, sem2)
        cp1.wait()
        cp2.wait()

        def comp_body(g, cnt):
            s16 = srcv[pl.ds(g * 16, 16)]
            dl = dstv[pl.ds(g * 16, 16)] - lo
            m = (dl >= 0) & (dl < tile)
            mi = jnp.where(m, jnp.int32(1), jnp.int32(0))
            pos = plsc.cumsum(mi) - 1 + cnt
            plsc.store_scatter(sbuf, [pos], s16, mask=m)
            plsc.store_scatter(dbuf, [pos], dl, mask=m)
            return cnt + jnp.sum(mi)

        cnt = lax.fori_loop(0, _CH // 16, comp_body, 0)
        z16 = jnp.zeros((16,), jnp.int32)
        t16 = jnp.full((16,), tile, jnp.int32)
        for j in range(_GK // 16):
            sbuf[pl.ds(cnt + j * 16, 16)] = z16
            dbuf[pl.ds(cnt + j * 16, 16)] = t16
        ngroups = (cnt + _GK - 1) // _GK
        lax.fori_loop(0, ngroups, group_body, lo)
        return lo

    def tile_body(tt, _):
        lo = (tt * _NC + c) * tile
        for r in range(nz):
            pltpu.sync_copy(zbuf, acc.at[pl.ds(row0 + r * 16, 16)])
        plsc.subcore_barrier()
        lax.fori_loop(0, n_chunk_iters, chunk_body, lo)
        plsc.subcore_barrier()
        outb = (tt * _NC + c) * acc_rows + row0
        for r in range(nz):
            pltpu.sync_copy(acc.at[pl.ds(row0 + r * 16, 16)],
                            out.at[pl.ds(outb + r * 16, 16)])
        plsc.subcore_barrier()
        return 0

    lax.fori_loop(0, tiles_per_core, tile_body, 0)


def _aggregate(hs, es, ed, src, dst, n_dst, heads, ch):
    hc = heads * ch
    w = hc + 16
    hsx = jnp.concatenate([hs, es], axis=1)
    tile, n_tiles, acc_rows = _pick_tile(n_dst, w)
    tiles_per_core = n_tiles // _NC
    edp = jnp.zeros((n_tiles * tile + 16, 16), jnp.float32).at[:n_dst].set(ed)

    e = src.shape[0]
    ep = -(-e // (_CH * _NS)) * (_CH * _NS)
    srcp = jnp.concatenate([src, jnp.zeros((ep - e,), jnp.int32)])
    dstp = jnp.concatenate(
        [dst, jnp.full((ep - e,), 1 << 30, jnp.int32)])
    n_chunk_iters = ep // (_CH * _NS)

    mesh = plsc.VectorSubcoreMesh(core_axis_name="c", subcore_axis_name="s")
    body = functools.partial(
        _sc_agg_body, heads=heads, ch=ch, tile=tile, acc_rows=acc_rows,
        tiles_per_core=tiles_per_core, n_chunk_iters=n_chunk_iters)

    def run(hsx_a, edp_a, srcp_a, dstp_a):
        def wrapped(hsx_r, edp_r, srcp_r, dstp_r, out_r, *scr):
            acc = scr[-1]
            body(hsx_r, edp_r, srcp_r, dstp_r, out_r, *scr[:-1], acc=acc)
        return pl.kernel(
            wrapped,
            out_type=jax.ShapeDtypeStruct((n_tiles * acc_rows, w),
                                          jnp.float32),
            mesh=mesh,
            compiler_params=pltpu.CompilerParams(
                needs_layout_passes=False, use_tc_tiling_on_sc=False),
            scratch_types=[
                pltpu.VMEM((_CH,), jnp.int32),
                pltpu.VMEM((_CH,), jnp.int32),
                pltpu.VMEM((_CH + 144,), jnp.int32),
                pltpu.VMEM((_CH + 144,), jnp.int32),
                pltpu.VMEM((_GK,), jnp.int32),
                pltpu.VMEM((_GK,), jnp.int32),
                pltpu.VMEM((_GK,), jnp.int32),
                pltpu.VMEM((_GK, w), jnp.float32),
                pltpu.VMEM((_GK, 16), jnp.float32),
                pltpu.VMEM((16, w), jnp.float32),
                pltpu.SemaphoreType.DMA,
                pltpu.SemaphoreType.DMA,
                pltpu.SemaphoreType.DMA,
                pltpu.SemaphoreType.DMA,
                pltpu.VMEM_SHARED((acc_rows, w), jnp.float32),
            ],
        )(hsx_a, edp_a, srcp_a, dstp_a)

    num = run(hsx, edp, srcp, dstp)
    num = num.reshape(n_tiles, acc_rows, w)[:, :tile]
    num = num.reshape(n_tiles * tile, w)[:n_dst]
    return num[:, :hc], num[:, hc:hc + heads]


def _finish(num, den, b, heads, ch):
    den = jnp.where(den > 0, den, 1.0)
    o = num.reshape(-1, heads, ch) / den[:, :, None]
    return o.mean(axis=1) + b


def _gat_edge(h_src, h_dst, src, dst, w1, a_s, a_d, b, heads, ch, same):
    if same:
        p = _gat_proj(h_src, w1, a_s, a_d, ("hs", "es", "ed"))
        hs, es, ed = p["hs"], p["es"], p["ed"]
    else:
        p = _gat_proj(h_src, w1, a_s, a_d, ("hs", "es"))
        hs, es = p["hs"], p["es"]
        ed = _gat_proj(h_dst, w1, a_s, a_d, ("ed",))["ed"]
    num, den = _aggregate(hs, es, ed, src, dst, h_dst.shape[0], heads, ch)
    return _finish(num, den, b, heads, ch)


def kernel(x_user, x_product, x_category, ei_prefers, ei_similar, ei_belongs, Wp_user, bp_user, Wd_user, bd_user, Wo_user, bo_user, Wp_product, bp_product, Wd_product, bd_product, Wo_product, bo_product, Wp_category, bp_category, Wd_category, bd_category, Wo_category, bo_category, W1_prefers, as1_prefers, ad1_prefers, b1_prefers, W2_prefers, as2_prefers, ad2_prefers, b2_prefers, W1_similar, as1_similar, ad1_similar, b1_similar, W2_similar, as2_similar, ad2_similar, b2_similar, W1_belongs, as1_belongs, ad1_belongs, b1_belongs, W2_belongs, as2_belongs, ad2_belongs, b2_belongs):
    n_prod = x_product.shape[0]
    lp = jnp.arange(n_prod, dtype=ei_similar.dtype)
    sim_src = jnp.concatenate([ei_similar[0], lp])
    sim_dst = jnp.concatenate([ei_similar[1], lp])

    h_u = _mm(x_user, Wp_user, bp_user, act="relu")
    h_p = _mm(x_product, Wp_product, bp_product, act="relu")
    h_c = _mm(x_category, Wp_category, bp_category, act="relu")

    o_pref = _gat_edge(h_u, h_p, ei_prefers[0], ei_prefers[1],
                       W1_prefers, as1_prefers, ad1_prefers, b1_prefers,
                       _HEADS, _HID, False)
    o_sim = _gat_edge(h_p, h_p, sim_src, sim_dst,
                      W1_similar, as1_similar, ad1_similar, b1_similar,
                      _HEADS, _HID, True)
    o_bel = _gat_edge(h_p, h_c, ei_belongs[0], ei_belongs[1],
                      W1_belongs, as1_belongs, ad1_belongs, b1_belongs,
                      _HEADS, _HID, False)
    h_p1 = jax.nn.relu(0.5 * (o_pref + o_sim))
    h_c1 = jax.nn.relu(o_bel)

    o_pref2 = _gat_edge(h_u, h_p1, ei_prefers[0], ei_prefers[1],
                        W2_prefers, as2_prefers, ad2_prefers, b2_prefers,
                        1, _OUT, False)
    o_sim2 = _gat_edge(h_p1, h_p1, sim_src, sim_dst,
                       W2_similar, as2_similar, ad2_similar, b2_similar,
                       1, _OUT, True)
    o_bel2 = _gat_edge(h_p1, h_c1, ei_belongs[0], ei_belongs[1],
                       W2_belongs, as2_belongs, ad2_belongs, b2_belongs,
                       1, _OUT, False)
    h_p2 = 0.5 * (o_pref2 + o_sim2)
    h_c2 = o_bel2

    wu = Wd_user @ Wo_user
    bu = bd_user @ Wo_user + bo_user
    out_u = _mm(h_u, wu, bu)
    out_p = _mm(h_p2, Wo_product, bo_product)
    out_c = _mm(h_c2, Wo_category, bo_category)
    return jnp.concatenate([out_u, out_p, out_c], axis=0)
